# Initial kernel scaffold; baseline (speedup 1.0000x reference)
import jax
import jax.numpy as jnp
from jax.experimental import pallas as pl


def kernel(x):
    raise NotImplementedError("write your pallas kernel here")


# speedup vs baseline: 635.7719x; 635.7719x over previous
import functools

import numpy as np
import jax
import jax.numpy as jnp
from jax import lax
from jax.experimental import pallas as pl
from jax.experimental.pallas import tpu as pltpu
from jax.experimental.pallas import tpu_sc as plsc

WIN = 1024
HOP = 256
NFFT = 1024
FBINS = 513
T = 4097
FP = 640
TB = 272
GRID = 16
TP = TB * GRID
M = TB + 8
CROWS = 4368
NW = 32
CHUNK = TP // NW
FB = 8
NB = CHUNK // FB


def _build_w() -> np.ndarray:
    n = np.arange(NFFT, dtype=np.int64)[:, None]
    k = np.arange(FP, dtype=np.int64)[None, :]
    ang = 2.0 * np.pi * ((n * k) % NFFT) / NFFT
    cos = np.cos(ang)
    sin = -np.sin(ang)
    cos[:, FBINS:] = 0.0
    sin[:, FBINS:] = 0.0
    hann = 0.5 - 0.5 * np.cos(2.0 * np.pi * np.arange(NFFT) / NFFT)
    w = np.concatenate([hann[:, None] * cos, hann[:, None] * sin], axis=1)
    return w.astype(np.float32)


_W = _build_w()


def _tc_body(c_ref, w_ref, vre_ref, vim_ref, kk_ref):
    i = pl.program_id(0)
    s = i * TB
    rows = c_ref[pl.ds(s, M + 8), :]
    frames = jnp.concatenate(
        [rows[0:M], rows[1:M + 1], rows[2:M + 2], rows[3:M + 3]], axis=1)
    spec = lax.dot_general(
        frames, w_ref[...], (((1,), (0,)), ((), ())),
        precision=lax.Precision.HIGHEST,
        preferred_element_type=jnp.float32)
    re = spec[:, :FP]
    im = spec[:, FP:]
    ph = jnp.arctan2(im, re)
    nxt = ph[1:TB + 1] - ph[0:TB]
    prv = jnp.concatenate([nxt[0:1], nxt[0:TB - 1]], axis=0)
    trow = s + lax.broadcasted_iota(jnp.int32, (TB, FP), 0)
    inst = jnp.where(trow == T - 1, prv, nxt)
    fint = lax.broadcasted_iota(jnp.int32, (TB, FP), 1)
    fcol = fint.astype(jnp.float32)
    k = (fcol + inst).astype(jnp.int32)
    real_f = fint < FBINS
    valid = (k >= 0) & (k < FBINS) & real_f
    kc = jnp.where(real_f, jnp.clip(k, 0, FBINS - 1), fint)
    rr = lax.broadcasted_iota(jnp.int32, (TB, FP), 0)
    vre_ref[...] = jnp.where(valid, re[0:TB], 0.0)
    vim_ref[...] = jnp.where(valid, im[0:TB], 0.0)
    kk_ref[...] = kc + (rr & (FB - 1)) * FP


def _tc_stage(c, w, interpret=False):
    return pl.pallas_call(
        _tc_body,
        grid=(GRID,),
        in_specs=[
            pl.BlockSpec((CROWS, 256), lambda i: (0, 0)),
            pl.BlockSpec((NFFT, 2 * FP), lambda i: (0, 0)),
        ],
        out_specs=[
            pl.BlockSpec((TB, FP), lambda i: (i, 0)),
            pl.BlockSpec((TB, FP), lambda i: (i, 0)),
            pl.BlockSpec((TB, FP), lambda i: (i, 0)),
        ],
        out_shape=[
            jax.ShapeDtypeStruct((TP, FP), jnp.float32),
            jax.ShapeDtypeStruct((TP, FP), jnp.float32),
            jax.ShapeDtypeStruct((TP, FP), jnp.int32),
        ],
        interpret=interpret,
    )(c, w)


BATCH = FB * FP
NVEC = BATCH // 16


def _sc_scatter_body(vre_hbm, vim_hbm, kk_hbm, outre_hbm, outim_hbm,
                     bre, bim, bk, are, aim, sem):
    wid = lax.axis_index("s") * 2 + lax.axis_index("c")
    base = wid * CHUNK * FP
    zero = jnp.zeros((16,), jnp.float32)

    def batch(it, carry):
        off = base + it * BATCH
        pltpu.sync_copy(vre_hbm.at[pl.ds(off, BATCH)], bre)
        pltpu.sync_copy(vim_hbm.at[pl.ds(off, BATCH)], bim)
        pltpu.sync_copy(kk_hbm.at[pl.ds(off, BATCH)], bk)
        for j in range(NVEC):
            sl = pl.ds(j * 16, 16)
            are[sl] = zero
            aim[sl] = zero
        for j in range(NVEC):
            sl = pl.ds(j * 16, 16)
            idx = bk[sl]
            plsc.addupdate_scatter(are, [idx], bre[sl])
            plsc.addupdate_scatter(aim, [idx], bim[sl])
        pltpu.sync_copy(are, outre_hbm.at[pl.ds(off, BATCH)])
        pltpu.sync_copy(aim, outim_hbm.at[pl.ds(off, BATCH)])
        return carry

    lax.fori_loop(0, NB, batch, 0)


@functools.cache
def _sc_scatter():
    mesh = plsc.VectorSubcoreMesh(core_axis_name="c", subcore_axis_name="s")
    return pl.kernel(
        _sc_scatter_body,
        out_type=[
            jax.ShapeDtypeStruct((TP * FP,), jnp.float32),
            jax.ShapeDtypeStruct((TP * FP,), jnp.float32),
        ],
        mesh=mesh,
        scratch_types=[
            pltpu.VMEM((BATCH,), jnp.float32),
            pltpu.VMEM((BATCH,), jnp.float32),
            pltpu.VMEM((BATCH,), jnp.int32),
            pltpu.VMEM((BATCH,), jnp.float32),
            pltpu.VMEM((BATCH,), jnp.float32),
            pltpu.SemaphoreType.DMA,
        ],
        compiler_params=pltpu.CompilerParams(
            use_tc_tiling_on_sc=False, needs_layout_passes=False),
    )


def kernel(x):
    xp = jnp.pad(x, (NFFT // 2, NFFT // 2), mode="reflect")
    c = jnp.zeros((CROWS, HOP), jnp.float32).at[:xp.shape[0] // HOP].set(
        xp.reshape(-1, HOP))
    vre, vim, kk = _tc_stage(c, jnp.asarray(_W))
    outre, outim = _sc_scatter()(
        vre.reshape(TP * FP), vim.reshape(TP * FP), kk.reshape(TP * FP))
    outre = outre.reshape(TP, FP)
    outim = outim.reshape(TP, FP)
    return lax.complex(outre[:T, :FBINS].T, outim[:T, :FBINS].T)
